# speedup vs baseline: 3.3299x; 3.3299x over previous
import functools

import jax
import jax.numpy as jnp
from jax import lax
from jax.experimental import pallas as pl
from jax.experimental.pallas import tpu as pltpu
from jax.experimental.pallas import tpu_sc as plsc

NC = 2
NS = 16
NW = NC * NS
CL = 128


@functools.cache
def _build(n_total: int, n_chunks: int, d: int):
    mesh = plsc.VectorSubcoreMesh(core_axis_name="c", subcore_axis_name="s")
    per_w = n_total // NW

    @functools.partial(
        pl.kernel,
        mesh=mesh,
        out_type=jax.ShapeDtypeStruct((n_total, d), jnp.float32),
        scratch_types=[
            pltpu.VMEM((n_chunks, CL), jnp.int32),
            pltpu.VMEM((2, CL, d), jnp.float32),
            pltpu.SemaphoreType.DMA,
            pltpu.SemaphoreType.DMA,
        ],
    )
    def gather_kernel(idx_hbm, table_hbm, out_hbm, idx_v, rows_v, sem0, sem1):
        wid = lax.axis_index("s") * NC + lax.axis_index("c")
        base = wid * per_w

        pltpu.sync_copy(idx_hbm.at[wid], idx_v)

        sems = (sem0, sem1)

        def gather(j, b):
            pltpu.async_copy(table_hbm.at[idx_v.at[j]], rows_v.at[b], sems[b])

        def wait(b):
            pltpu.make_async_copy(
                table_hbm.at[idx_v.at[0]], rows_v.at[b], sems[b]
            ).wait()

        def copy_out(j, b):
            pltpu.sync_copy(rows_v.at[b], out_hbm.at[pl.ds(base + j * CL, CL)])

        gather(0, 0)
        gather(1, 1)

        def body(jo, carry):
            j = jo * 2
            wait(0)
            copy_out(j, 0)
            gather(j + 2, 0)
            wait(1)
            copy_out(j + 1, 1)
            gather(j + 3, 1)
            return carry

        lax.fori_loop(0, n_chunks // 2 - 1, body, 0)

        j_tail = n_chunks - 2
        wait(0)
        copy_out(j_tail, 0)
        wait(1)
        copy_out(j_tail + 1, 1)

    return gather_kernel


def kernel(token_ids, W):
    b, l = token_ids.shape
    d = W.shape[1]
    n_total = b * l
    idx = token_ids.reshape(-1).astype(jnp.int32)
    n_chunks = n_total // (NW * CL)
    idx3 = idx.reshape(NW, n_chunks, CL)
    out = _build(n_total, n_chunks, d)(idx3, W)
    return out.reshape(b, l, d)
